# baseline (speedup 1.0000x reference)
import functools

import jax
import jax.numpy as jnp
from jax import lax
from jax.experimental import pallas as pl
from jax.experimental.pallas import tpu as pltpu
from jax.experimental.pallas import tpu_sc as plsc

N_TOK = 16384
D = 256
V = 8192
TM = 512
TW = 2736
N_WIN = 3
N_TT = N_TOK // TM
V_PAD = TW * N_WIN


def _argmin_body(z_ref, z2_ref, e_ref, idx_out, loss_out,
                 accv_s, acci_s, accx_s, lsum_s):
    j = pl.program_id(0)
    i = pl.program_id(1)
    rows = pl.ds(i * TM, TM)

    z = z_ref[...]
    e = e_ref[...]
    s = lax.dot_general(z.astype(jnp.bfloat16), e.astype(jnp.bfloat16),
                        (((1,), (1,)), ((), ())),
                        preferred_element_type=jnp.float32)
    e2 = jnp.sum(e * e, axis=1)
    z2 = z2_ref[...]
    cols = lax.broadcasted_iota(jnp.int32, (TM, TW), 1) + j * TW
    t = jnp.where(cols >= V, jnp.float32(jnp.inf), z2 + e2[None, :])
    d = t - 2.0 * s

    wmin = jnp.min(d, axis=1, keepdims=True)
    warg = jnp.min(jnp.where(d == wmin, cols, jnp.int32(2**30)),
                   axis=1, keepdims=True)

    @pl.when(j == 0)
    def _first():
        accv_s[rows, :] = wmin.astype(jnp.bfloat16).astype(jnp.float32)
        acci_s[rows, :] = warg
        accx_s[rows, :] = wmin

    @pl.when(j > 0)
    def _fold():
        av = accv_s[rows, :]
        ai = acci_s[rows, :]
        ax = accx_s[rows, :]
        lt = wmin < av
        take = lt | ((wmin == av) & (warg < ai))
        acci_s[rows, :] = jnp.where(take, warg, ai)
        accx_s[rows, :] = jnp.where(take, wmin, ax)
        accv_s[rows, :] = jnp.where(lt, wmin, av).astype(
            jnp.bfloat16).astype(jnp.float32)

    @pl.when(j == N_WIN - 1)
    def _finish():
        idx_out[...] = jnp.reshape(acci_s[rows, :], (TM,))

        @pl.when(i == 0)
        def _z():
            lsum_s[0] = 0.0
        lsum_s[0] += jnp.sum(accx_s[rows, :])

        @pl.when(i == N_TT - 1)
        def _w():
            loss_out[0, 0] = 2.0 * lsum_s[0] / jnp.float32(N_TOK * D)


def _argmin_call(z_flat, z2, e_pad):
    return pl.pallas_call(
        _argmin_body,
        grid=(N_WIN, N_TT),
        in_specs=[
            pl.BlockSpec((TM, D), lambda j, i: (i, 0)),
            pl.BlockSpec((TM, 1), lambda j, i: (i, 0)),
            pl.BlockSpec((TW, D), lambda j, i: (j, 0)),
        ],
        out_specs=[
            pl.BlockSpec((TM,), lambda j, i: (i,)),
            pl.BlockSpec(memory_space=pltpu.SMEM),
        ],
        out_shape=[
            jax.ShapeDtypeStruct((N_TOK,), jnp.int32),
            jax.ShapeDtypeStruct((1, 1), jnp.float32),
        ],
        scratch_shapes=[
            pltpu.VMEM((N_TOK, 1), jnp.float32),
            pltpu.VMEM((N_TOK, 1), jnp.int32),
            pltpu.VMEM((N_TOK, 1), jnp.float32),
            pltpu.SMEM((1,), jnp.float32),
        ],
    )(z_flat, z2, e_pad)


_NW = 32
_BPW = N_TOK // _NW
_CHUNK = 128
_NCHUNK = _BPW // _CHUNK


def _gather_body(idx_hbm, table_hbm, out_hbm, idx_v, buf0, buf1, sem0, sem1):
    wid = lax.axis_index("s") * 2 + lax.axis_index("c")
    base = wid * _BPW
    pltpu.sync_copy(idx_hbm.at[pl.ds(base, _BPW)], idx_v)
    bufs = (buf0, buf1)
    sems = (sem0, sem1)

    def start(c):
        return pltpu.async_copy(
            table_hbm.at[idx_v.at[pl.ds(c * _CHUNK, _CHUNK)]],
            bufs[c % 2], sems[c % 2])

    cur = start(0)
    for c in range(_NCHUNK):
        nxt = start(c + 1) if c + 1 < _NCHUNK else None
        cur.wait()
        pltpu.sync_copy(bufs[c % 2],
                        out_hbm.at[pl.ds(base + c * _CHUNK, _CHUNK)])
        cur = nxt


def _gather_call(idx, emb):
    mesh = plsc.VectorSubcoreMesh(core_axis_name="c", subcore_axis_name="s")
    k = functools.partial(
        pl.kernel, mesh=mesh,
        out_type=jax.ShapeDtypeStruct((N_TOK, D), jnp.float32),
        scratch_types=[
            pltpu.VMEM((_BPW,), jnp.int32),
            pltpu.VMEM((_CHUNK, D), jnp.float32),
            pltpu.VMEM((_CHUNK, D), jnp.float32),
            pltpu.SemaphoreType.DMA,
            pltpu.SemaphoreType.DMA,
        ],
    )(_gather_body)
    return k(idx, emb)


def kernel(z_e, embedding_weight):
    z_flat = z_e.reshape(N_TOK, D)
    z2 = jnp.sum(z_e ** 2, axis=-1).reshape(N_TOK, 1)
    e_pad = jnp.pad(embedding_weight, ((0, V_PAD - V), (0, 0)))
    idx, loss = _argmin_call(z_flat, z2, e_pad)
    z_q = _gather_call(idx, embedding_weight)
    return (z_e, jnp.reshape(loss, ()), z_q.reshape(z_e.shape))


# speedup vs baseline: 1.5000x; 1.5000x over previous
import functools

import jax
import jax.numpy as jnp
from jax import lax
from jax.experimental import pallas as pl
from jax.experimental.pallas import tpu as pltpu
from jax.experimental.pallas import tpu_sc as plsc

N_TOK = 16384
D = 256
V = 8192
TM = 2048
TW = 2736
TWB = 2816
N_WIN = 3
N_TT = N_TOK // TM


def _argmin_body(z_ref, z2_ref, e_ref, e2_ref, idx_out, loss_out,
                 accv_s, acci_s, accx_s, lsum_s):
    j = pl.program_id(0)
    i = pl.program_id(1)
    rows = pl.ds(i * TM, TM)

    zb = z_ref[...]
    eb = e_ref[...]
    s2 = lax.dot_general(zb, eb, (((1,), (1,)), ((), ())),
                         preferred_element_type=jnp.float32)
    z2 = z2_ref[...]
    m = (z2 + e2_ref[0:1, 0:128]) + s2[:, 0:128]
    mi = jnp.zeros((TM, 128), jnp.int32)
    for c in range(1, TWB // 128):
        dch = ((z2 + e2_ref[0:1, c * 128:(c + 1) * 128])
               + s2[:, c * 128:(c + 1) * 128])
        lt = dch < m
        m = jnp.where(lt, dch, m)
        mi = jnp.where(lt, jnp.int32(c), mi)

    lane = lax.broadcasted_iota(jnp.int32, (TM, 128), 1)
    gidx = mi * 128 + lane
    wmin = jnp.min(m, axis=1, keepdims=True)
    warg = jnp.min(jnp.where(m == wmin, gidx, jnp.int32(2**30)),
                   axis=1, keepdims=True) + j * TW

    @pl.when(j == 0)
    def _first():
        accv_s[rows, :] = wmin.astype(jnp.bfloat16).astype(jnp.float32)
        acci_s[rows, :] = warg
        accx_s[rows, :] = wmin

    @pl.when(j > 0)
    def _fold():
        av = accv_s[rows, :]
        ai = acci_s[rows, :]
        ax = accx_s[rows, :]
        lt = wmin < av
        take = lt | ((wmin == av) & (warg < ai))
        acci_s[rows, :] = jnp.where(take, warg, ai)
        accx_s[rows, :] = jnp.where(take, wmin, ax)
        accv_s[rows, :] = jnp.where(lt, wmin, av).astype(
            jnp.bfloat16).astype(jnp.float32)

    @pl.when(j == N_WIN - 1)
    def _finish():
        idx_out[...] = jnp.reshape(acci_s[rows, :], (TM,))

        @pl.when(i == 0)
        def _z():
            lsum_s[0] = 0.0
        lsum_s[0] += jnp.sum(accx_s[rows, :])

        @pl.when(i == N_TT - 1)
        def _w():
            loss_out[0, 0] = 2.0 * lsum_s[0] / jnp.float32(N_TOK * D)


def _argmin_call(z_bf, z2, e_bf, e2m):
    return pl.pallas_call(
        _argmin_body,
        grid=(N_WIN, N_TT),
        in_specs=[
            pl.BlockSpec((TM, D), lambda j, i: (i, 0)),
            pl.BlockSpec((TM, 1), lambda j, i: (i, 0)),
            pl.BlockSpec((TWB, D), lambda j, i: (j, 0)),
            pl.BlockSpec((8, TWB), lambda j, i: (j, 0)),
        ],
        out_specs=[
            pl.BlockSpec((TM,), lambda j, i: (i,)),
            pl.BlockSpec(memory_space=pltpu.SMEM),
        ],
        out_shape=[
            jax.ShapeDtypeStruct((N_TOK,), jnp.int32),
            jax.ShapeDtypeStruct((1, 1), jnp.float32),
        ],
        scratch_shapes=[
            pltpu.VMEM((N_TOK, 1), jnp.float32),
            pltpu.VMEM((N_TOK, 1), jnp.int32),
            pltpu.VMEM((N_TOK, 1), jnp.float32),
            pltpu.SMEM((1,), jnp.float32),
        ],
    )(z_bf, z2, e_bf, e2m)


_NW = 32
_BPW = N_TOK // _NW
_CHUNK = 128
_NCHUNK = _BPW // _CHUNK


def _gather_body(idx_hbm, table_hbm, out_hbm, idx_v, buf0, buf1, sem0, sem1):
    wid = lax.axis_index("s") * 2 + lax.axis_index("c")
    base = wid * _BPW
    pltpu.sync_copy(idx_hbm.at[pl.ds(base, _BPW)], idx_v)
    bufs = (buf0, buf1)
    sems = (sem0, sem1)

    def start(c):
        return pltpu.async_copy(
            table_hbm.at[idx_v.at[pl.ds(c * _CHUNK, _CHUNK)]],
            bufs[c % 2], sems[c % 2])

    cur = start(0)
    for c in range(_NCHUNK):
        nxt = start(c + 1) if c + 1 < _NCHUNK else None
        cur.wait()
        pltpu.sync_copy(bufs[c % 2],
                        out_hbm.at[pl.ds(base + c * _CHUNK, _CHUNK)])
        cur = nxt


def _gather_call(idx, emb):
    mesh = plsc.VectorSubcoreMesh(core_axis_name="c", subcore_axis_name="s")
    k = functools.partial(
        pl.kernel, mesh=mesh,
        out_type=jax.ShapeDtypeStruct((N_TOK, D), jnp.float32),
        scratch_types=[
            pltpu.VMEM((_BPW,), jnp.int32),
            pltpu.VMEM((_CHUNK, D), jnp.float32),
            pltpu.VMEM((_CHUNK, D), jnp.float32),
            pltpu.SemaphoreType.DMA,
            pltpu.SemaphoreType.DMA,
        ],
    )(_gather_body)
    return k(idx, emb)


def kernel(z_e, embedding_weight):
    z_flat = z_e.reshape(N_TOK, D)
    z2 = jnp.sum(z_e ** 2, axis=-1).reshape(N_TOK, 1)
    e2 = jnp.sum(embedding_weight ** 2, axis=1)

    z_bf = z_flat.astype(jnp.bfloat16)
    em2 = (-2.0 * embedding_weight).astype(jnp.bfloat16)
    e_bf = jnp.concatenate([
        em2[0:TWB],
        em2[TW:TW + TWB],
        jnp.pad(em2[2 * TW:V], ((0, TWB - (V - 2 * TW)), (0, 0))),
    ])

    inf = jnp.float32(jnp.inf)
    loc = jnp.arange(TWB)
    rows = []
    for w in range(N_WIN):
        gcol = loc + w * TW
        seg = lax.dynamic_slice(jnp.pad(e2, (0, 2 * TWB), constant_values=0.0),
                                (w * TW,), (TWB,))
        rows.append(jnp.where((loc < TW) & (gcol < V), seg, inf))
    e2m = jnp.tile(jnp.stack(rows)[:, None, :], (1, 8, 1)).reshape(
        N_WIN * 8, TWB)

    idx, loss = _argmin_call(z_bf, z2, e_bf, e2m)
    z_q = _gather_call(idx, embedding_weight)
    return (z_e, jnp.reshape(loss, ()), z_q.reshape(z_e.shape))


# speedup vs baseline: 2.0982x; 1.3988x over previous
import functools

import jax
import jax.numpy as jnp
from jax import lax
from jax.experimental import pallas as pl
from jax.experimental.pallas import tpu as pltpu
from jax.experimental.pallas import tpu_sc as plsc

N_TOK = 16384
D = 256
V = 8192
TM = 2048
TW = 2736
V_PAD = 8208
N_WIN = 3
N_TT = N_TOK // TM
N_CH = TW // 8


def _argmin_body(z_ref, z2_ref, e_ref, e2_ref, idx_out, loss_out,
                 accv_s, acci_s, accx_s, lsum_s):
    j = pl.program_id(0)
    i = pl.program_id(1)

    zb = z_ref[...]
    eb = e_ref[...]
    s2 = lax.dot_general(eb, zb, (((1,), (1,)), ((), ())),
                         preferred_element_type=jnp.float32)
    z2 = z2_ref[...]
    m = (z2 + e2_ref[0:8, 0:1]) + s2[0:8, :]
    mi = jnp.zeros((8, TM), jnp.int32)
    for c in range(1, N_CH):
        dch = (z2 + e2_ref[c * 8:(c + 1) * 8, 0:1]) + s2[c * 8:(c + 1) * 8, :]
        lt = dch < m
        m = jnp.where(lt, dch, m)
        mi = jnp.where(lt, jnp.int32(c), mi)

    subl = lax.broadcasted_iota(jnp.int32, (8, TM), 0)
    gidx = mi * 8 + subl
    wmin = jnp.min(m, axis=0, keepdims=True)
    warg = jnp.min(jnp.where(m == wmin, gidx, jnp.int32(2**30)),
                   axis=0, keepdims=True) + j * TW

    row = pl.ds(i, 1)

    @pl.when(j == 0)
    def _first():
        accv_s[row, :] = wmin.astype(jnp.bfloat16).astype(jnp.float32)
        acci_s[row, :] = warg
        accx_s[row, :] = wmin

    @pl.when(j > 0)
    def _fold():
        av = accv_s[row, :]
        ai = acci_s[row, :]
        ax = accx_s[row, :]
        lt = wmin < av
        take = lt | ((wmin == av) & (warg < ai))
        acci_s[row, :] = jnp.where(take, warg, ai)
        accx_s[row, :] = jnp.where(take, wmin, ax)
        accv_s[row, :] = jnp.where(lt, wmin, av).astype(
            jnp.bfloat16).astype(jnp.float32)

    @pl.when(j == N_WIN - 1)
    def _finish():
        idx_out[...] = jnp.reshape(acci_s[row, :], (TM,))

        @pl.when(i == 0)
        def _z():
            lsum_s[0] = 0.0
        lsum_s[0] += jnp.sum(accx_s[row, :])

        @pl.when(i == N_TT - 1)
        def _w():
            loss_out[0, 0] = 2.0 * lsum_s[0] / jnp.float32(N_TOK * D)


def _argmin_call(z_bf, z2r, e_bf, e2c):
    return pl.pallas_call(
        _argmin_body,
        grid=(N_WIN, N_TT),
        in_specs=[
            pl.BlockSpec((TM, D), lambda j, i: (i, 0)),
            pl.BlockSpec((1, TM), lambda j, i: (0, i)),
            pl.BlockSpec((TW, D), lambda j, i: (j, 0)),
            pl.BlockSpec((TW, 1), lambda j, i: (j, 0)),
        ],
        out_specs=[
            pl.BlockSpec((TM,), lambda j, i: (i,)),
            pl.BlockSpec(memory_space=pltpu.SMEM),
        ],
        out_shape=[
            jax.ShapeDtypeStruct((N_TOK,), jnp.int32),
            jax.ShapeDtypeStruct((1, 1), jnp.float32),
        ],
        scratch_shapes=[
            pltpu.VMEM((N_TT, TM), jnp.float32),
            pltpu.VMEM((N_TT, TM), jnp.int32),
            pltpu.VMEM((N_TT, TM), jnp.float32),
            pltpu.SMEM((1,), jnp.float32),
        ],
    )(z_bf, z2r, e_bf, e2c)


_NW = 32
_BPW = N_TOK // _NW
_CHUNK = 128
_NCHUNK = _BPW // _CHUNK


def _gather_body(idx_hbm, table_hbm, out_hbm, idx_v, buf0, buf1, sem0, sem1):
    wid = lax.axis_index("s") * 2 + lax.axis_index("c")
    base = wid * _BPW
    pltpu.sync_copy(idx_hbm.at[pl.ds(base, _BPW)], idx_v)
    bufs = (buf0, buf1)
    sems = (sem0, sem1)

    def start(c):
        return pltpu.async_copy(
            table_hbm.at[idx_v.at[pl.ds(c * _CHUNK, _CHUNK)]],
            bufs[c % 2], sems[c % 2])

    cur = start(0)
    for c in range(_NCHUNK):
        nxt = start(c + 1) if c + 1 < _NCHUNK else None
        cur.wait()
        pltpu.sync_copy(bufs[c % 2],
                        out_hbm.at[pl.ds(base + c * _CHUNK, _CHUNK)])
        cur = nxt


def _gather_call(idx, emb):
    mesh = plsc.VectorSubcoreMesh(core_axis_name="c", subcore_axis_name="s")
    k = functools.partial(
        pl.kernel, mesh=mesh,
        out_type=jax.ShapeDtypeStruct((N_TOK, D), jnp.float32),
        scratch_types=[
            pltpu.VMEM((_BPW,), jnp.int32),
            pltpu.VMEM((_CHUNK, D), jnp.float32),
            pltpu.VMEM((_CHUNK, D), jnp.float32),
            pltpu.SemaphoreType.DMA,
            pltpu.SemaphoreType.DMA,
        ],
    )(_gather_body)
    return k(idx, emb)


def kernel(z_e, embedding_weight):
    z2r = jnp.sum(z_e ** 2, axis=-1).reshape(1, N_TOK)
    e2 = jnp.sum(embedding_weight ** 2, axis=1)

    z_bf = z_e.reshape(N_TOK, D).astype(jnp.bfloat16)
    e_bf = jnp.pad((-2.0 * embedding_weight).astype(jnp.bfloat16),
                   ((0, V_PAD - V), (0, 0)))
    e2c = jnp.pad(e2, (0, V_PAD - V),
                  constant_values=jnp.inf).reshape(V_PAD, 1)

    idx, loss = _argmin_call(z_bf, z2r, e_bf, e2c)
    z_q = _gather_call(idx, embedding_weight)
    return (z_e, jnp.reshape(loss, ()), z_q.reshape(z_e.shape))


# speedup vs baseline: 2.1275x; 1.0140x over previous
import functools

import jax
import jax.numpy as jnp
from jax import lax
from jax.experimental import pallas as pl
from jax.experimental.pallas import tpu as pltpu
from jax.experimental.pallas import tpu_sc as plsc

N_TOK = 16384
D = 256
V = 8192
TM = 4096
TW = 2736
V_PAD = 8208
N_WIN = 3
N_TT = N_TOK // TM
N_CH = TW // 8


def _argmin_body(z_ref, z2_ref, e_ref, e2_ref, idx_out, loss_out,
                 accv_s, acci_s, accx_s, lsum_s):
    j = pl.program_id(0)
    i = pl.program_id(1)

    zb = z_ref[...]
    eb = e_ref[...]
    s2 = lax.dot_general(eb, zb, (((1,), (1,)), ((), ())),
                         preferred_element_type=jnp.float32)
    z2 = z2_ref[...]
    m = (z2 + e2_ref[0:8, 0:1]) + s2[0:8, :]
    mi = jnp.zeros((8, TM), jnp.int32)
    for c in range(1, N_CH):
        dch = (z2 + e2_ref[c * 8:(c + 1) * 8, 0:1]) + s2[c * 8:(c + 1) * 8, :]
        lt = dch < m
        m = jnp.where(lt, dch, m)
        mi = jnp.where(lt, jnp.int32(c), mi)

    subl = lax.broadcasted_iota(jnp.int32, (8, TM), 0)
    gidx = mi * 8 + subl
    wmin = jnp.min(m, axis=0, keepdims=True)
    warg = jnp.min(jnp.where(m == wmin, gidx, jnp.int32(2**30)),
                   axis=0, keepdims=True) + j * TW

    row = pl.ds(i, 1)

    @pl.when(j == 0)
    def _first():
        accv_s[row, :] = wmin.astype(jnp.bfloat16).astype(jnp.float32)
        acci_s[row, :] = warg
        accx_s[row, :] = wmin

    @pl.when(j > 0)
    def _fold():
        av = accv_s[row, :]
        ai = acci_s[row, :]
        ax = accx_s[row, :]
        lt = wmin < av
        take = lt | ((wmin == av) & (warg < ai))
        acci_s[row, :] = jnp.where(take, warg, ai)
        accx_s[row, :] = jnp.where(take, wmin, ax)
        accv_s[row, :] = jnp.where(lt, wmin, av).astype(
            jnp.bfloat16).astype(jnp.float32)

    @pl.when(j == N_WIN - 1)
    def _finish():
        idx_out[...] = jnp.reshape(acci_s[row, :], (TM,))

        @pl.when(i == 0)
        def _z():
            lsum_s[0] = 0.0
        lsum_s[0] += jnp.sum(accx_s[row, :])

        @pl.when(i == N_TT - 1)
        def _w():
            loss_out[0, 0] = 2.0 * lsum_s[0] / jnp.float32(N_TOK * D)


def _argmin_call(z_bf, z2r, e_bf, e2c):
    return pl.pallas_call(
        _argmin_body,
        grid=(N_WIN, N_TT),
        in_specs=[
            pl.BlockSpec((TM, D), lambda j, i: (i, 0)),
            pl.BlockSpec((1, TM), lambda j, i: (0, i)),
            pl.BlockSpec((TW, D), lambda j, i: (j, 0)),
            pl.BlockSpec((TW, 1), lambda j, i: (j, 0)),
        ],
        out_specs=[
            pl.BlockSpec((TM,), lambda j, i: (i,)),
            pl.BlockSpec(memory_space=pltpu.SMEM),
        ],
        out_shape=[
            jax.ShapeDtypeStruct((N_TOK,), jnp.int32),
            jax.ShapeDtypeStruct((1, 1), jnp.float32),
        ],
        scratch_shapes=[
            pltpu.VMEM((N_TT, TM), jnp.float32),
            pltpu.VMEM((N_TT, TM), jnp.int32),
            pltpu.VMEM((N_TT, TM), jnp.float32),
            pltpu.SMEM((1,), jnp.float32),
        ],
    )(z_bf, z2r, e_bf, e2c)


_NW = 32
_BPW = N_TOK // _NW
_CHUNK = 128
_NCHUNK = _BPW // _CHUNK


def _gather_body(idx_hbm, table_hbm, out_hbm, idx_v, buf0, buf1, sem0, sem1):
    wid = lax.axis_index("s") * 2 + lax.axis_index("c")
    base = wid * _BPW
    pltpu.sync_copy(idx_hbm.at[pl.ds(base, _BPW)], idx_v)
    bufs = (buf0, buf1)
    sems = (sem0, sem1)

    def start(c):
        return pltpu.async_copy(
            table_hbm.at[idx_v.at[pl.ds(c * _CHUNK, _CHUNK)]],
            bufs[c % 2], sems[c % 2])

    cur = start(0)
    for c in range(_NCHUNK):
        nxt = start(c + 1) if c + 1 < _NCHUNK else None
        cur.wait()
        pltpu.sync_copy(bufs[c % 2],
                        out_hbm.at[pl.ds(base + c * _CHUNK, _CHUNK)])
        cur = nxt


def _gather_call(idx, emb):
    mesh = plsc.VectorSubcoreMesh(core_axis_name="c", subcore_axis_name="s")
    k = functools.partial(
        pl.kernel, mesh=mesh,
        out_type=jax.ShapeDtypeStruct((N_TOK, D), jnp.float32),
        scratch_types=[
            pltpu.VMEM((_BPW,), jnp.int32),
            pltpu.VMEM((_CHUNK, D), jnp.float32),
            pltpu.VMEM((_CHUNK, D), jnp.float32),
            pltpu.SemaphoreType.DMA,
            pltpu.SemaphoreType.DMA,
        ],
    )(_gather_body)
    return k(idx, emb)


def kernel(z_e, embedding_weight):
    z2r = jnp.sum(z_e ** 2, axis=-1).reshape(1, N_TOK)
    e2 = jnp.sum(embedding_weight ** 2, axis=1)

    z_bf = z_e.reshape(N_TOK, D).astype(jnp.bfloat16)
    e_bf = jnp.pad((-2.0 * embedding_weight).astype(jnp.bfloat16),
                   ((0, V_PAD - V), (0, 0)))
    e2c = jnp.pad(e2, (0, V_PAD - V),
                  constant_values=jnp.inf).reshape(V_PAD, 1)

    idx, loss = _argmin_call(z_bf, z2r, e_bf, e2c)
    z_q = _gather_call(idx, embedding_weight)
    return (z_e, jnp.reshape(loss, ()), z_q.reshape(z_e.shape))
